# Initial kernel scaffold; baseline (speedup 1.0000x reference)
import jax
import jax.numpy as jnp
from jax.experimental import pallas as pl


def kernel(x, w1, b1, w2, b2, ws, bs):
    raise NotImplementedError("write your pallas kernel here")


# speedup vs baseline: 1.0314x; 1.0314x over previous
import functools
import math

import jax
import jax.numpy as jnp
from jax.experimental import pallas as pl
from jax.experimental.pallas import tpu as pltpu


def _pick_batch_tile(n):
    divisors = [d for d in range(1, min(n, 8) + 1) if n % d == 0]
    multi = [d for d in divisors if n // d >= 2]
    return max(multi) if multi else max(divisors)


def _attn_kernel(x_ref, w1_ref, b1_ref, w2_ref, b2_ref, wt_ref, wm_ref,
                 bs_ref, o_ref, pad_ref, *, NT, H, W, PAD):
    HW = H * W
    x = x_ref[...]

    mean_c = jnp.mean(x, axis=2, keepdims=True)
    h = jnp.sum(w1_ref[...][None, :, :] * mean_c, axis=1, keepdims=True)
    h = jnp.maximum(h + b1_ref[...], 0.0)
    logit = jnp.sum(w2_ref[...][None, :, :] * h, axis=2, keepdims=True)
    logit = logit + b2_ref[...]
    ch_att = 1.0 / (1.0 + jnp.exp(-logit))
    xs = x * ch_att

    avg_r = jnp.mean(xs, axis=1)
    max_r = jnp.max(xs, axis=1)

    zeros_halo = jnp.zeros((2 * NT, PAD), jnp.float32)
    pad_ref[:, pl.ds(0, PAD)] = zeros_halo
    pad_ref[:, pl.ds(PAD + HW, PAD)] = zeros_halo
    pad_ref[pl.ds(0, NT), pl.ds(PAD, HW)] = avg_r
    pad_ref[pl.ds(NT, NT), pl.ds(PAD, HW)] = max_r

    u = [pad_ref[:, pl.ds(PAD + dh * W, HW)] for dh in range(-3, 4)]

    wt = wt_ref[...]

    y = jnp.zeros((2 * NT, HW), jnp.float32)
    for dw in range(-3, 4):
        s = None
        for dh in range(-3, 4):
            t = (dh + 3) * 7 + (dw + 3)
            term = u[dh + 3] * wt[:, t:t + 1]
            s = term if s is None else s + term
        if dw == 0:
            y = y + s
        else:
            rolled = pltpu.roll(s, (-dw) % HW, axis=1)
            y = y + rolled * wm_ref[dw + 3:dw + 4, :]

    conv = y[0:NT, :] + y[NT:2 * NT, :] + bs_ref[0, 0]
    sp_att = 1.0 / (1.0 + jnp.exp(-conv))

    o_ref[...] = (xs * sp_att[:, None, :]).astype(o_ref.dtype)


def _forward(x, w1, b1, w2, b2, ws, bs):
    N, C, H, W = x.shape
    C8 = w1.shape[0]
    HW = H * W

    NT = _pick_batch_tile(N)
    grid = N // NT

    x_rows = x.reshape(N, C, HW)
    w1t = w1.reshape(C8, C).T
    b1r = b1.reshape(1, C8)
    w2m = w2.reshape(C, C8)
    b2c = b2.reshape(C, 1)
    ws2 = ws.reshape(2, 7 * 7)
    w_tab = jnp.concatenate(
        [jnp.broadcast_to(ws2[0:1], (NT, 49)),
         jnp.broadcast_to(ws2[1:2], (NT, 49))], axis=0)
    bs2 = bs.reshape(1, 1)

    w_idx = jax.lax.broadcasted_iota(jnp.int32, (8, HW), 1) % W
    dw_col = jnp.array([-3, -2, -1, 0, 1, 2, 3, 0],
                       dtype=jnp.int32).reshape(8, 1)
    wmask = ((w_idx + dw_col >= 0) & (w_idx + dw_col < W)).astype(jnp.float32)

    PAD = ((3 * W + 3 + 127) // 128) * 128
    PADW = HW + 2 * PAD

    flops = int(N * (6 * C * HW + 4 * 49 * 2 * HW + 4 * C * C8))
    bytes_accessed = int(4 * (2 * N * C * HW + 2 * C * C8 + C + C8 + 2 * 49 + 1))

    _kernel_fn = functools.partial(_attn_kernel, NT=NT, H=H, W=W, PAD=PAD)
    out_rows = pl.pallas_call(
        _kernel_fn,
        out_shape=jax.ShapeDtypeStruct((N, C, HW), jnp.float32),
        grid_spec=pltpu.PrefetchScalarGridSpec(
            num_scalar_prefetch=0,
            grid=(grid,),
            in_specs=[
                pl.BlockSpec((NT, C, HW), lambda n: (n, 0, 0)),
                pl.BlockSpec((C, C8), lambda n: (0, 0)),
                pl.BlockSpec((1, C8), lambda n: (0, 0)),
                pl.BlockSpec((C, C8), lambda n: (0, 0)),
                pl.BlockSpec((C, 1), lambda n: (0, 0)),
                pl.BlockSpec((2 * NT, 49), lambda n: (0, 0)),
                pl.BlockSpec((8, HW), lambda n: (0, 0)),
                pl.BlockSpec(memory_space=pltpu.MemorySpace.SMEM),
            ],
            out_specs=pl.BlockSpec((NT, C, HW), lambda n: (n, 0, 0)),
            scratch_shapes=[
                pltpu.VMEM((2 * NT, PADW), jnp.float32),
            ],
        ),
        compiler_params=pltpu.CompilerParams(
            dimension_semantics=("parallel",),
            vmem_limit_bytes=32 * 1024 * 1024,
        ),
        cost_estimate=pl.CostEstimate(
            flops=flops,
            transcendentals=int(N * (C + HW)),
            bytes_accessed=bytes_accessed,
        ),
    )(x_rows, w1t, b1r, w2m, b2c, w_tab, wmask, bs2)

    return out_rows.reshape(N, C, H, W)


def kernel(x, w1, b1, w2, b2, ws, bs):
    return _forward(x, w1, b1, w2, b2, ws, bs)


# speedup vs baseline: 3.5206x; 3.4135x over previous
import functools

import jax
import jax.numpy as jnp
from jax.experimental import pallas as pl
from jax.experimental.pallas import tpu as pltpu


def _attn_kernel(x_hbm, w1_ref, b1_ref, w2_ref, b2_ref, wt_s, bs_s, o_hbm,
                 xv, mA, mB, in_sems, out_sems, *, C, H, W, N, CB):
    NCHUNK = C // CB

    def in_copy(k):
        return pltpu.make_async_copy(
            x_hbm.at[pl.ds(k * CB, CB)], xv.at[pl.ds(k * CB, CB)],
            in_sems.at[k])

    def out_copy(k):
        return pltpu.make_async_copy(
            xv.at[pl.ds(k * CB, CB)], o_hbm.at[pl.ds(k * CB, CB)],
            out_sems.at[k])

    for k in range(NCHUNK):
        in_copy(k).start()

    inv_hw = 1.0 / (H * W)
    mean_rows = []
    for k in range(NCHUNK):
        in_copy(k).wait()
        for c in range(k * CB, (k + 1) * CB):
            s = jnp.sum(xv[c], axis=(0, 1), keepdims=True)
            mean_rows.append(s[0] * inv_hw)
    mean = jnp.concatenate(mean_rows, axis=0)

    h = jnp.dot(w1_ref[...], mean, preferred_element_type=jnp.float32)
    h = jnp.maximum(h + b1_ref[...], 0.0)
    logit = jnp.dot(w2_ref[...], h, preferred_element_type=jnp.float32)
    logit = logit + b2_ref[...]
    ch_att = 1.0 / (1.0 + jnp.exp(-logit))

    HB = H // 2
    for hh in range(2):
        h0 = hh * HB
        accA = None
        accM = None
        for c in range(C):
            catt_c = ch_att[c].reshape(1, 1, N)
            xs = xv[c, pl.ds(h0, HB)] * catt_c
            xv[c, pl.ds(h0, HB)] = xs
            accA = xs if accA is None else accA + xs
            accM = xs if accM is None else jnp.maximum(accM, xs)
        mA[pl.ds(3 + h0, HB)] = accA * (1.0 / C)
        mB[pl.ds(3 + h0, HB)] = accM

    zrow = jnp.zeros((3, W, N), jnp.float32)
    mA[pl.ds(0, 3)] = zrow
    mA[pl.ds(3 + H, 3)] = zrow
    mB[pl.ds(0, 3)] = zrow
    mB[pl.ds(3 + H, 3)] = zrow

    HCH = 8
    satt_chunks = []
    for h0 in range(0, H, HCH):
        winA = mA[pl.ds(h0, HCH + 6)]
        winB = mB[pl.ds(h0, HCH + 6)]
        y = None
        for dw in range(-3, 4):
            z = None
            for dh in range(-3, 4):
                t = (dh + 3) * 7 + (dw + 3)
                term = (winA[dh + 3:dh + 3 + HCH] * wt_s[0, t]
                        + winB[dh + 3:dh + 3 + HCH] * wt_s[1, t])
                z = term if z is None else z + term
            if dw > 0:
                z = jnp.concatenate(
                    [z[:, dw:, :], jnp.zeros((HCH, dw, N), jnp.float32)],
                    axis=1)
            elif dw < 0:
                z = jnp.concatenate(
                    [jnp.zeros((HCH, -dw, N), jnp.float32), z[:, :W + dw, :]],
                    axis=1)
            y = z if y is None else y + z
        conv = y + bs_s[0, 0]
        satt_chunks.append(1.0 / (1.0 + jnp.exp(-conv)))
    satt = jnp.concatenate(satt_chunks, axis=0)

    for k in range(NCHUNK):
        for c in range(k * CB, (k + 1) * CB):
            xv[c] = xv[c] * satt
        out_copy(k).start()
    for k in range(NCHUNK):
        out_copy(k).wait()


def _forward(x, w1, b1, w2, b2, ws, bs):
    N, C, H, W = x.shape
    C8 = w1.shape[0]

    x_t = jnp.transpose(x, (1, 2, 3, 0))

    w1v = w1.reshape(C8, C)
    b1b = jnp.broadcast_to(b1.reshape(C8, 1), (C8, N))
    w2v = w2.reshape(C, C8)
    b2b = jnp.broadcast_to(b2.reshape(C, 1), (C, N))
    wt2 = ws.reshape(2, 49)
    bs2 = bs.reshape(1, 1)

    CB = 8
    flops = int(N * (6 * C * H * W + 4 * 49 * 2 * H * W + 4 * C * C8))
    bytes_accessed = int(4 * (2 * N * C * H * W))

    _kernel_fn = functools.partial(_attn_kernel, C=C, H=H, W=W, N=N, CB=CB)
    out_t = pl.pallas_call(
        _kernel_fn,
        out_shape=jax.ShapeDtypeStruct((C, H, W, N), jnp.float32),
        in_specs=[
            pl.BlockSpec(memory_space=pl.ANY),
            pl.BlockSpec(memory_space=pltpu.MemorySpace.VMEM),
            pl.BlockSpec(memory_space=pltpu.MemorySpace.VMEM),
            pl.BlockSpec(memory_space=pltpu.MemorySpace.VMEM),
            pl.BlockSpec(memory_space=pltpu.MemorySpace.VMEM),
            pl.BlockSpec(memory_space=pltpu.MemorySpace.SMEM),
            pl.BlockSpec(memory_space=pltpu.MemorySpace.SMEM),
        ],
        out_specs=pl.BlockSpec(memory_space=pl.ANY),
        scratch_shapes=[
            pltpu.VMEM((C, H, W, N), jnp.float32),
            pltpu.VMEM((H + 6, W, N), jnp.float32),
            pltpu.VMEM((H + 6, W, N), jnp.float32),
            pltpu.SemaphoreType.DMA((C // CB,)),
            pltpu.SemaphoreType.DMA((C // CB,)),
        ],
        compiler_params=pltpu.CompilerParams(
            vmem_limit_bytes=60 * 1024 * 1024,
        ),
        cost_estimate=pl.CostEstimate(
            flops=flops,
            transcendentals=int(N * (C + H * W)),
            bytes_accessed=bytes_accessed,
        ),
    )(x_t, w1v, b1b, w2v, b2b, wt2, bs2)

    return jnp.transpose(out_t, (3, 0, 1, 2))


def kernel(x, w1, b1, w2, b2, ws, bs):
    return _forward(x, w1, b1, w2, b2, ws, bs)
